# baseline (speedup 1.0000x reference)
import dataclasses
import functools

import jax
import jax.numpy as jnp
from jax import lax
from jax.experimental import pallas as pl
from jax.experimental.pallas import tpu as pltpu
from jax.experimental.pallas import tpu_sc as plsc

_LANES = 16


def _sc_compiler_params():
    cp = pltpu.CompilerParams()
    if "needs_layout_passes" in pltpu.CompilerParams.__dataclass_fields__:
        cp = dataclasses.replace(cp, needs_layout_passes=False)
    return cp


def _diffuse_sc(x2, n2, t, tab_a, tab_s):
    NROW, B = x2.shape
    T = tab_a.shape[0]
    NW = 32
    rows_pw = NROW // NW
    CH = 64
    nchunks = rows_pw // CH
    nm = B // _LANES
    mesh = plsc.VectorSubcoreMesh(core_axis_name="c", subcore_axis_name="s")
    num_cores = mesh.num_cores

    @functools.partial(
        pl.kernel,
        out_type=jax.ShapeDtypeStruct((NROW, B), jnp.float32),
        mesh=mesh,
        scratch_types=[
            pltpu.VMEM((2, CH, B), jnp.float32),
            pltpu.VMEM((2, CH, B), jnp.float32),
            pltpu.VMEM((2, CH, B), jnp.float32),
            pltpu.VMEM((B,), jnp.int32),
            pltpu.VMEM((T,), jnp.float32),
            pltpu.VMEM((T,), jnp.float32),
            pltpu.SemaphoreType.DMA,
            pltpu.SemaphoreType.DMA,
            pltpu.SemaphoreType.DMA,
            pltpu.SemaphoreType.DMA,
            pltpu.SemaphoreType.DMA,
            pltpu.SemaphoreType.DMA,
            pltpu.SemaphoreType.DMA,
        ],
        compiler_params=_sc_compiler_params(),
    )
    def body(x_hbm, n_hbm, t_hbm, ta_hbm, ts_hbm, o_hbm,
             xb, nb, ob, idx_v, ta_v, ts_v,
             si0, si1, so0, so1, sg0, sg1, sg2):
        wid = lax.axis_index("s") * num_cores + lax.axis_index("c")
        base = wid * rows_pw
        si = (si0, si1)
        so = (so0, so1)

        def rows(c):
            return pl.ds(base + c * CH, CH)

        h_in = {}
        for c in range(2):
            h_in[("x", c)] = pltpu.async_copy(x_hbm.at[rows(c)], xb.at[c], si[c])
            h_in[("n", c)] = pltpu.async_copy(n_hbm.at[rows(c)], nb.at[c], si[c])

        g0 = pltpu.async_copy(t_hbm, idx_v, sg0)
        g1 = pltpu.async_copy(ta_hbm, ta_v, sg1)
        g2 = pltpu.async_copy(ts_hbm, ts_v, sg2)
        g0.wait()
        g1.wait()
        g2.wait()

        A = []
        S = []
        for m in range(nm):
            iv = idx_v[pl.ds(_LANES * m, _LANES)]
            A.append(plsc.load_gather(ta_v, [iv]))
            S.append(plsc.load_gather(ts_v, [iv]))

        @pl.loop(0, nchunks, step=2)
        def _(k):
            for b in range(2):
                c = k + b
                pltpu.make_async_copy(x_hbm.at[rows(c)], xb.at[b], si[b]).wait()
                pltpu.make_async_copy(n_hbm.at[rows(c)], nb.at[b], si[b]).wait()

                @pl.when(k >= 2)
                def _():
                    pltpu.make_async_copy(
                        ob.at[b], o_hbm.at[rows(c)], so[b]).wait()

                @plsc.parallel_loop(0, CH, step=8, unroll=2)
                def _(j, b=b):
                    for d in range(8):
                        for m in range(nm):
                            sl = (b, j + d, pl.ds(_LANES * m, _LANES))
                            ob[sl] = A[m] * xb[sl] + S[m] * nb[sl]

                pltpu.async_copy(ob.at[b], o_hbm.at[rows(c)], so[b])

                @pl.when(c + 2 < nchunks)
                def _():
                    pltpu.async_copy(x_hbm.at[rows(c + 2)], xb.at[b], si[b])
                    pltpu.async_copy(n_hbm.at[rows(c + 2)], nb.at[b], si[b])

        for b in range(2):
            pltpu.make_async_copy(ob.at[b], o_hbm.at[rows(0)], so[b]).wait()

    return body(x2, n2, t, tab_a, tab_s)


def kernel(x0, t, noise, sqrt_alphas_bar, sqrt_one_minus_alphas_bar):
    B, C, H, W = x0.shape
    x2 = jnp.transpose(x0, (1, 2, 3, 0)).reshape(C * H * W, B)
    n2 = jnp.transpose(noise, (1, 2, 3, 0)).reshape(C * H * W, B)
    out2 = _diffuse_sc(x2, n2, t, sqrt_alphas_bar, sqrt_one_minus_alphas_bar)
    return jnp.transpose(out2.reshape(C, H, W, B), (3, 0, 1, 2))


# speedup vs baseline: 1.7269x; 1.7269x over previous
import dataclasses
import functools

import jax
import jax.numpy as jnp
from jax import lax
from jax.experimental import pallas as pl
from jax.experimental.pallas import tpu as pltpu
from jax.experimental.pallas import tpu_sc as plsc

_LANES = 16


def _sc_compiler_params():
    cp = pltpu.CompilerParams()
    if "needs_layout_passes" in pltpu.CompilerParams.__dataclass_fields__:
        cp = dataclasses.replace(cp, needs_layout_passes=False)
    return cp


def _gather_coeffs_sc(t, tab_a, tab_s):
    B = t.shape[0]
    T = tab_a.shape[0]
    mesh = plsc.ScalarSubcoreMesh(axis_name="c", num_cores=2)

    @functools.partial(
        pl.kernel,
        out_type=(
            jax.ShapeDtypeStruct((B,), jnp.float32),
            jax.ShapeDtypeStruct((B,), jnp.float32),
        ),
        mesh=mesh,
        scratch_types=[
            pltpu.SMEM((B,), jnp.int32),
            pltpu.SMEM((T,), jnp.float32),
            pltpu.SMEM((B,), jnp.float32),
            pltpu.SemaphoreType.DMA,
            pltpu.SemaphoreType.DMA,
        ],
        compiler_params=_sc_compiler_params(),
    )
    def gather_kernel(t_hbm, ta_hbm, ts_hbm, oa_hbm, os_hbm,
                      idx_s, tab_s_ref, out_s, sem0, sem1):
        cid = lax.axis_index("c")
        c0 = pltpu.async_copy(t_hbm, idx_s, sem0)

        @pl.when(cid == 0)
        def _():
            pltpu.async_copy(ta_hbm, tab_s_ref, sem1).wait()

        @pl.when(cid == 1)
        def _():
            pltpu.async_copy(ts_hbm, tab_s_ref, sem1).wait()

        c0.wait()

        @pl.loop(0, B)
        def _(i):
            out_s[i] = tab_s_ref[idx_s[i]]

        @pl.when(cid == 0)
        def _():
            pltpu.async_copy(out_s, oa_hbm, sem1).wait()

        @pl.when(cid == 1)
        def _():
            pltpu.async_copy(out_s, os_hbm, sem1).wait()

    return gather_kernel(t, tab_a, tab_s)


def _combine_body(a_ref, s_ref, x_ref, n_ref, o_ref):
    o_ref[...] = a_ref[...] * x_ref[...] + s_ref[...] * n_ref[...]


def _combine_tc(x2, n2, a, s, row_block):
    NR, B = x2.shape
    RB = row_block
    blk = (RB, B)
    idx = lambda j: (j, 0)
    cidx = lambda j: (0,)
    return pl.pallas_call(
        _combine_body,
        grid=(NR // RB,),
        in_specs=[
            pl.BlockSpec((B,), cidx),
            pl.BlockSpec((B,), cidx),
            pl.BlockSpec(blk, idx),
            pl.BlockSpec(blk, idx),
        ],
        out_specs=pl.BlockSpec(blk, idx),
        out_shape=jax.ShapeDtypeStruct((NR, B), jnp.float32),
        compiler_params=pltpu.CompilerParams(
            dimension_semantics=("arbitrary",),
        ),
    )(a, s, x2, n2)


def kernel(x0, t, noise, sqrt_alphas_bar, sqrt_one_minus_alphas_bar):
    B, C, H, W = x0.shape
    a, s = _gather_coeffs_sc(t, sqrt_alphas_bar, sqrt_one_minus_alphas_bar)
    x2 = jnp.transpose(x0, (1, 2, 3, 0)).reshape(C * H * W, B)
    n2 = jnp.transpose(noise, (1, 2, 3, 0)).reshape(C * H * W, B)
    out2 = _combine_tc(x2, n2, a, s, row_block=1024)
    return jnp.transpose(out2.reshape(C, H, W, B), (3, 0, 1, 2))


# speedup vs baseline: 1.8527x; 1.0728x over previous
import dataclasses
import functools

import jax
import jax.numpy as jnp
from jax import lax
from jax.experimental import pallas as pl
from jax.experimental.pallas import tpu as pltpu
from jax.experimental.pallas import tpu_sc as plsc

_LANES = 16


def _sc_compiler_params():
    cp = pltpu.CompilerParams()
    if "needs_layout_passes" in pltpu.CompilerParams.__dataclass_fields__:
        cp = dataclasses.replace(cp, needs_layout_passes=False)
    return cp


def _gather_coeffs_sc(t, tab_a, tab_s):
    B = t.shape[0]
    T = tab_a.shape[0]
    mesh = plsc.ScalarSubcoreMesh(axis_name="c", num_cores=2)

    @functools.partial(
        pl.kernel,
        out_type=(
            jax.ShapeDtypeStruct((B,), jnp.float32),
            jax.ShapeDtypeStruct((B,), jnp.float32),
        ),
        mesh=mesh,
        scratch_types=[
            pltpu.SMEM((B,), jnp.int32),
            pltpu.SMEM((T,), jnp.float32),
            pltpu.SMEM((B,), jnp.float32),
            pltpu.SemaphoreType.DMA,
            pltpu.SemaphoreType.DMA,
        ],
        compiler_params=_sc_compiler_params(),
    )
    def gather_kernel(t_hbm, ta_hbm, ts_hbm, oa_hbm, os_hbm,
                      idx_s, tab_s_ref, out_s, sem0, sem1):
        cid = lax.axis_index("c")
        c0 = pltpu.async_copy(t_hbm, idx_s, sem0)

        @pl.when(cid == 0)
        def _():
            pltpu.async_copy(ta_hbm, tab_s_ref, sem1).wait()

        @pl.when(cid == 1)
        def _():
            pltpu.async_copy(ts_hbm, tab_s_ref, sem1).wait()

        c0.wait()

        @pl.loop(0, B)
        def _(i):
            out_s[i] = tab_s_ref[idx_s[i]]

        @pl.when(cid == 0)
        def _():
            pltpu.async_copy(out_s, oa_hbm, sem1).wait()

        @pl.when(cid == 1)
        def _():
            pltpu.async_copy(out_s, os_hbm, sem1).wait()

    return gather_kernel(t, tab_a, tab_s)


def _combine_body(a_ref, s_ref, x_ref, n_ref, o_ref):
    o_ref[...] = a_ref[...] * x_ref[...] + s_ref[...] * n_ref[...]


def _combine_tc(x2, n2, a, s, row_block):
    NR, B = x2.shape
    RB = row_block
    blk = (RB, B)
    idx = lambda j: (j, 0)
    cidx = lambda j: (0,)
    return pl.pallas_call(
        _combine_body,
        grid=(NR // RB,),
        in_specs=[
            pl.BlockSpec((B,), cidx),
            pl.BlockSpec((B,), cidx),
            pl.BlockSpec(blk, idx),
            pl.BlockSpec(blk, idx),
        ],
        out_specs=pl.BlockSpec(blk, idx),
        out_shape=jax.ShapeDtypeStruct((NR, B), jnp.float32),
        compiler_params=pltpu.CompilerParams(
            dimension_semantics=("arbitrary",),
        ),
    )(a, s, x2, n2)


def kernel(x0, t, noise, sqrt_alphas_bar, sqrt_one_minus_alphas_bar):
    B, C, H, W = x0.shape
    a, s = _gather_coeffs_sc(t, sqrt_alphas_bar, sqrt_one_minus_alphas_bar)
    x2 = jnp.transpose(x0, (1, 2, 3, 0)).reshape(C * H * W, B)
    n2 = jnp.transpose(noise, (1, 2, 3, 0)).reshape(C * H * W, B)
    out2 = _combine_tc(x2, n2, a, s, row_block=4096)
    return jnp.transpose(out2.reshape(C, H, W, B), (3, 0, 1, 2))


# speedup vs baseline: 1.8923x; 1.0214x over previous
import dataclasses
import functools

import jax
import jax.numpy as jnp
from jax import lax
from jax.experimental import pallas as pl
from jax.experimental.pallas import tpu as pltpu
from jax.experimental.pallas import tpu_sc as plsc

_LANES = 16


def _sc_compiler_params():
    cp = pltpu.CompilerParams()
    if "needs_layout_passes" in pltpu.CompilerParams.__dataclass_fields__:
        cp = dataclasses.replace(cp, needs_layout_passes=False)
    return cp


def _gather_coeffs_sc(t, tab_a, tab_s):
    B = t.shape[0]
    T = tab_a.shape[0]
    mesh = plsc.ScalarSubcoreMesh(axis_name="c", num_cores=2)

    @functools.partial(
        pl.kernel,
        out_type=(
            jax.ShapeDtypeStruct((B,), jnp.float32),
            jax.ShapeDtypeStruct((B,), jnp.float32),
        ),
        mesh=mesh,
        scratch_types=[
            pltpu.SMEM((B,), jnp.int32),
            pltpu.SMEM((T,), jnp.float32),
            pltpu.SMEM((B,), jnp.float32),
            pltpu.SemaphoreType.DMA,
            pltpu.SemaphoreType.DMA,
        ],
        compiler_params=_sc_compiler_params(),
    )
    def gather_kernel(t_hbm, ta_hbm, ts_hbm, oa_hbm, os_hbm,
                      idx_s, tab_s_ref, out_s, sem0, sem1):
        cid = lax.axis_index("c")
        c0 = pltpu.async_copy(t_hbm, idx_s, sem0)

        @pl.when(cid == 0)
        def _():
            pltpu.async_copy(ta_hbm, tab_s_ref, sem1).wait()

        @pl.when(cid == 1)
        def _():
            pltpu.async_copy(ts_hbm, tab_s_ref, sem1).wait()

        c0.wait()

        @pl.loop(0, B)
        def _(i):
            out_s[i] = tab_s_ref[idx_s[i]]

        @pl.when(cid == 0)
        def _():
            pltpu.async_copy(out_s, oa_hbm, sem1).wait()

        @pl.when(cid == 1)
        def _():
            pltpu.async_copy(out_s, os_hbm, sem1).wait()

    return gather_kernel(t, tab_a, tab_s)


def _combine_body(a_ref, s_ref, x_ref, n_ref, o_ref):
    o_ref[...] = a_ref[...] * x_ref[...] + s_ref[...] * n_ref[...]


def _combine_tc(xt, nt, a, s, h_block):
    C, H, W, B = xt.shape
    HB = h_block
    blk = (C, HB, W, B)
    idx = lambda j: (0, j, 0, 0)
    cidx = lambda j: (0,)
    return pl.pallas_call(
        _combine_body,
        grid=(H // HB,),
        in_specs=[
            pl.BlockSpec((B,), cidx),
            pl.BlockSpec((B,), cidx),
            pl.BlockSpec(blk, idx),
            pl.BlockSpec(blk, idx),
        ],
        out_specs=pl.BlockSpec(blk, idx),
        out_shape=jax.ShapeDtypeStruct((C, H, W, B), jnp.float32),
        compiler_params=pltpu.CompilerParams(
            dimension_semantics=("arbitrary",),
        ),
    )(a, s, xt, nt)


def kernel(x0, t, noise, sqrt_alphas_bar, sqrt_one_minus_alphas_bar):
    a, s = _gather_coeffs_sc(t, sqrt_alphas_bar, sqrt_one_minus_alphas_bar)
    xt = jnp.transpose(x0, (1, 2, 3, 0))
    nt = jnp.transpose(noise, (1, 2, 3, 0))
    out_t = _combine_tc(xt, nt, a, s, h_block=16)
    return jnp.transpose(out_t, (3, 0, 1, 2))
